# baseline (speedup 1.0000x reference)
import jax
import jax.numpy as jnp
from jax import lax
from jax.experimental import pallas as pl
from jax.experimental.pallas import tpu as pltpu
from jax.experimental.pallas import tpu_sc as plsc

B, L, D, V = 4096, 200, 128, 100000
NC, NS, LANES = 2, 16, 16
NW = NC * NS
BPW = B // NW
NBUF = 3
HI16 = jnp.int32(-65536)


def _emb_body(tb1_hbm, xt_hbm, pe_hbm, out_hbm, idx_v, pe_v, inbufs, outbufs,
              semg, sems):
    tb_hbm = tb1_hbm
    wid = lax.axis_index("s") * NC + lax.axis_index("c")
    b0 = wid * BPW

    pltpu.sync_copy(xt_hbm.at[:, pl.ds(b0, BPW)], idx_v)
    pltpu.sync_copy(pe_hbm, pe_v)

    def gather_fire(l, j):
        pltpu.async_copy(tb_hbm.at[idx_v.at[l]], inbufs[j], semg[j])

    def gather_wait(l, j):
        pltpu.make_async_copy(tb_hbm.at[idx_v.at[l]], inbufs[j], semg[j]).wait()

    def store_fire(l, j):
        pltpu.async_copy(outbufs[j], out_hbm.at[pl.ds(b0, BPW), l], sems[j])

    def store_wait(l, j):
        pltpu.make_async_copy(outbufs[j], out_hbm.at[pl.ds(b0, BPW), l],
                              sems[j]).wait()

    lane = lax.iota(jnp.int32, LANES)
    cols = [c * 32 + 2 * lane for c in range(D // 32)]

    def add_pe(l, j):
        inb, outb = inbufs[j], outbufs[j]
        pevs = [pe_v[l, pl.ds(k * LANES, LANES)] for k in range(D // LANES)]

        @plsc.parallel_loop(0, BPW, 1, unroll=4)
        def _body(b):
            bvec = jnp.full((LANES,), b, jnp.int32)
            for c in range(D // 32):
                ab32 = inb[b, pl.ds(c * LANES, LANES)]
                lo = lax.bitcast_convert_type(ab32 << 16, jnp.float32)
                hi = lax.bitcast_convert_type(ab32 & HI16, jnp.float32)
                plsc.store_scatter(outb, [bvec, cols[c]], lo + pevs[2 * c])
                plsc.store_scatter(outb, [bvec, cols[c] + 1], hi + pevs[2 * c + 1])

    def step(l, jj, first, last):
        gather_wait(l, jj)
        if not first:
            store_wait(l - NBUF, jj)
        add_pe(l, jj)
        store_fire(l, jj)
        if not last:
            gather_fire(l + NBUF, jj)

    for j in range(NBUF):
        gather_fire(j, j)

    for l in range(NBUF):
        step(l, l, True, False)

    def outer(i, carry):
        base = i * NBUF
        for jj in range(NBUF):
            step(base + jj, jj, False, False)
        return carry

    lax.fori_loop(1, 65, outer, 0)

    step(195, 0, False, False)
    step(196, 1, False, False)
    for l in range(197, 200):
        step(l, l % NBUF, False, True)
    for l in range(197, 200):
        store_wait(l, l % NBUF)


def kernel(x, table, pe):
    xt = x.T
    u = lax.bitcast_convert_type(table, jnp.uint32).reshape(V * D)
    r = (u + jnp.uint32(0x7FFF) + ((u >> 16) & jnp.uint32(1))) >> 16
    tb = lax.bitcast_convert_type(
        r[0::2] | (r[1::2] << 16), jnp.int32).reshape(V, D // 2)
    pe_shuf = (pe.reshape(L, D // 32, LANES, 2)
               .transpose(0, 1, 3, 2)
               .reshape(L, D))
    run = pl.kernel(
        _emb_body,
        out_type=jax.ShapeDtypeStruct((B, L, D), jnp.float32),
        mesh=plsc.VectorSubcoreMesh(core_axis_name="c", subcore_axis_name="s"),
        compiler_params=pltpu.CompilerParams(use_tc_tiling_on_sc=False,
                                             needs_layout_passes=False),
        scratch_types=[
            pltpu.VMEM((L, BPW), jnp.int32),
            pltpu.VMEM((L, D), jnp.float32),
            [pltpu.VMEM((BPW, D // 2), jnp.int32) for _ in range(NBUF)],
            [pltpu.VMEM((BPW, D), jnp.float32) for _ in range(NBUF)],
            [pltpu.SemaphoreType.DMA for _ in range(NBUF)],
            [pltpu.SemaphoreType.DMA for _ in range(NBUF)],
        ],
    )
    return run(tb, xt, pe_shuf)


# speedup vs baseline: 8.1872x; 8.1872x over previous
import jax
import jax.numpy as jnp
from jax import lax
from jax.experimental import pallas as pl
from jax.experimental.pallas import tpu as pltpu
from jax.experimental.pallas import tpu_sc as plsc

B, L, D, V = 4096, 200, 128, 100000
NC, NS, LANES = 2, 16, 16
NW = NC * NS
BPW = B // NW
NCHUNK = D // LANES
NBUF = 4


def _emb_body(xt_hbm, table_hbm, pe_hbm, out_hbm, idx_v, pe_v, bufs, semg, sems):
    wid = lax.axis_index("s") * NC + lax.axis_index("c")
    b0 = wid * BPW

    pltpu.sync_copy(xt_hbm.at[:, pl.ds(b0, BPW)], idx_v)
    pltpu.sync_copy(pe_hbm, pe_v)

    def gather_fire(l, j):
        pltpu.async_copy(table_hbm.at[idx_v.at[l]], bufs[j], semg[j])

    def gather_wait(l, j):
        pltpu.make_async_copy(table_hbm.at[idx_v.at[l]], bufs[j], semg[j]).wait()

    def store_fire(l, j):
        pltpu.async_copy(bufs[j], out_hbm.at[pl.ds(b0, BPW), l], sems[j])

    def store_wait(l, j):
        pltpu.make_async_copy(bufs[j], out_hbm.at[pl.ds(b0, BPW), l], sems[j]).wait()

    def add_pe(l, j):
        buf = bufs[j]
        pevs = [pe_v[l, pl.ds(c * LANES, LANES)] for c in range(NCHUNK)]

        @plsc.parallel_loop(0, BPW, 1, unroll=4)
        def _body(b):
            for c in range(NCHUNK):
                sl = pl.ds(c * LANES, LANES)
                buf[b, sl] = buf[b, sl] + pevs[c]

    for j in range(NBUF - 1):
        gather_fire(j, j)

    def outer(i, carry):
        base = i * NBUF
        for jj in range(NBUF):
            l = base + jj
            gather_wait(l, jj)
            add_pe(l, jj)
            store_fire(l, jj)
            jp = (jj - 1) % NBUF
            store_wait(l - 1, jp)
            gather_fire(l + NBUF - 1, jp)
        return carry

    for jj in range(NBUF):
        l = jj
        gather_wait(l, jj)
        add_pe(l, jj)
        store_fire(l, jj)
        jp = (jj - 1) % NBUF
        if jj > 0:
            store_wait(l - 1, jp)
        gather_fire(l + NBUF - 1, jp)

    lax.fori_loop(1, L // NBUF - 1, outer, 0)

    base = L - NBUF
    for jj in range(NBUF):
        l = base + jj
        gather_wait(l, jj)
        add_pe(l, jj)
        store_fire(l, jj)
        jp = (jj - 1) % NBUF
        store_wait(l - 1, jp)
        if l + NBUF - 1 < L:
            gather_fire(l + NBUF - 1, jp)
    store_wait(L - 1, (NBUF - 1) % NBUF)


def kernel(x, table, pe):
    xt = x.T
    pe2 = pe.reshape(L, D)
    run = pl.kernel(
        _emb_body,
        out_type=jax.ShapeDtypeStruct((B, L, D), jnp.float32),
        mesh=plsc.VectorSubcoreMesh(core_axis_name="c", subcore_axis_name="s"),
        scratch_types=[
            pltpu.VMEM((L, BPW), jnp.int32),
            pltpu.VMEM((L, D), jnp.float32),
            [pltpu.VMEM((BPW, D), jnp.float32) for _ in range(NBUF)],
            [pltpu.SemaphoreType.DMA for _ in range(NBUF)],
            [pltpu.SemaphoreType.DMA for _ in range(NBUF)],
        ],
    )
    return run(xt, table, pe2)
